# baseline (speedup 1.0000x reference)
import functools

import jax
import jax.numpy as jnp
from jax import lax
from jax.experimental import pallas as pl
from jax.experimental.pallas import tpu as pltpu
from jax.experimental.pallas import tpu_sc as plsc

N = 10000
E = 320000
D = 128
NSUB = 16
NCORE = 2
HALF = D // 2
RPT = N // NSUB
RCH = 125
CH = 128
EPAD = 20480 * NSUB
EPT = EPAD // NSUB
EPW = E // (NSUB * NCORE)
DCH = 2000


def _mesh():
    return plsc.VectorSubcoreMesh(core_axis_name="c", subcore_axis_name="s")


_SC_PARAMS = pltpu.CompilerParams(
    use_tc_tiling_on_sc=False, needs_layout_passes=False)


def _deg_body(dsta, dstb, dstc, wexp, parts, acc, dvm, wvm):
    c = lax.axis_index("c")
    s = lax.axis_index("s")
    wid = s * NCORE + c
    ebase = wid * EPW
    zv = jnp.zeros((16,), jnp.float32)
    ones = jnp.ones((16,), jnp.float32)
    for a, dste in enumerate((dsta, dstb, dstc)):
        def zb(i, _):
            acc[pl.ds(i * 16, 16)] = zv
            return 0
        lax.fori_loop(0, N // 16, zb, 0)
        for chn in range(EPW // DCH):
            off = ebase + chn * DCH
            pltpu.sync_copy(dste.at[pl.ds(off, DCH)], dvm)
            if a == 2:
                pltpu.sync_copy(wexp.at[pl.ds(off, DCH)], wvm)

            def sca(k, _):
                idx = dvm[pl.ds(k * 16, 16)]
                vals = wvm[pl.ds(k * 16, 16)] if a == 2 else ones
                plsc.addupdate_scatter(acc, [idx], vals)
                return 0
            lax.fori_loop(0, DCH // 16, sca, 0)
        pltpu.sync_copy(acc, parts.at[a, wid])


def _sc_deg(dsta, dstb, dstc, wexp):
    k = pl.kernel(
        _deg_body,
        mesh=_mesh(),
        out_type=jax.ShapeDtypeStruct((3, NSUB * NCORE, N), jnp.float32),
        scratch_types=[
            pltpu.VMEM((N,), jnp.float32),
            pltpu.VMEM((DCH,), jnp.int32),
            pltpu.VMEM((DCH,), jnp.float32),
        ],
        compiler_params=_SC_PARAMS,
    )
    return k(dsta, dstb, dstc, wexp)


KB = 5
SETS = 2
NQ = KB * SETS
CPT = EPT // CH
NIT = CPT // NQ


def _agg_body(nrow, hp0, hp1, edata, zeros, out, spm_acc, ebuf, rows,
              esem, gsem0, gsem1, ssem0, ssem1):
    scale = nrow == 3
    c = lax.axis_index("c")
    s = lax.axis_index("s")
    row0 = s * RPT
    col0 = c * HALF
    gsem = (gsem0, gsem1)
    ssem = (ssem0, ssem1)
    pltpu.sync_copy(zeros, spm_acc.at[pl.ds(row0, RPT)])
    plsc.subcore_barrier()

    cbase = s * CPT

    def gathers(p):
        def fire(hp):
            for k in range(KB):
                q = p * KB + k
                pltpu.async_copy(hp.at[ebuf.at[q, 0]], rows.at[q], gsem[p])

        @pl.when(c == 0)
        def _():
            fire(hp0)

        @pl.when(c == 1)
        def _():
            fire(hp1)

    def drain_gathers(p):
        def drain(hp):
            for k in range(KB):
                q = p * KB + k
                pltpu.make_async_copy(hp.at[ebuf.at[q, 0]], rows.at[q],
                                      gsem[p]).wait()

        @pl.when(c == 0)
        def _():
            drain(hp0)

        @pl.when(c == 1)
        def _():
            drain(hp1)

    def scale_rows(p):
        for k in range(KB):
            q = p * KB + k

            def sg(g, _):
                wvec = plsc.bitcast(ebuf[q, 2, pl.ds(g * 16, 16)],
                                    jnp.float32)
                for lane in range(16):
                    kk = g * 16 + lane
                    wk = wvec[lane]
                    for f in range(HALF // 16):
                        rows[q, kk, pl.ds(f * 16, 16)] = (
                            rows[q, kk, pl.ds(f * 16, 16)] * wk)
                return 0
            lax.fori_loop(0, CH // 16, sg, 0)

    def it(j, _):
        cid0 = cbase + j * NQ
        pltpu.sync_copy(edata.at[pl.ds(cid0, NQ)], ebuf)
        for p in range(SETS):
            gathers(p)
        for p in range(SETS):
            drain_gathers(p)
            if scale:
                scale_rows(p)
            for k in range(KB):
                q = p * KB + k
                pltpu.async_copy(rows.at[q], spm_acc.at[ebuf.at[q, 1]],
                                 ssem[p], add=True)
        for p in range(SETS):
            for k in range(KB):
                q = p * KB + k
                pltpu.make_async_copy(rows.at[q], spm_acc.at[ebuf.at[q, 1]],
                                      ssem[p]).wait()
        return 0
    lax.fori_loop(0, NIT, it, 0)
    plsc.subcore_barrier()
    pltpu.sync_copy(spm_acc.at[pl.ds(row0, RPT)],
                    out.at[pl.ds(row0, RPT), pl.ds(col0, HALF)])


def _sc_agg(hp0, hp1, edata, zeros, nrow):
    body = functools.partial(_agg_body, nrow)
    k = pl.kernel(
        body,
        mesh=_mesh(),
        out_type=jax.ShapeDtypeStruct((N, D), jnp.float32),
        scratch_types=[
            pltpu.VMEM_SHARED((N + 16, HALF), jnp.float32),
            pltpu.VMEM((NQ, nrow, CH), jnp.int32),
            pltpu.VMEM((NQ, CH, HALF), jnp.float32),
            pltpu.SemaphoreType.DMA,
            pltpu.SemaphoreType.DMA,
            pltpu.SemaphoreType.DMA,
            pltpu.SemaphoreType.DMA,
            pltpu.SemaphoreType.DMA,
        ],
        compiler_params=_SC_PARAMS,
    )
    return k(hp0, hp1, edata, zeros)


def _elu(x):
    return jnp.where(x > 0, x, jnp.exp(x) - 1.0)


def _dis_body(parts, out):
    deg = 1.0 + jnp.sum(parts[...], axis=1)
    out[...] = lax.rsqrt(deg)


def _tc_dis(parts):
    return pl.pallas_call(
        _dis_body,
        out_shape=jax.ShapeDtypeStruct((3, N), jnp.float32),
    )(parts)


def _pre_body(x, w, dis, out0, out1):
    h = jnp.dot(x[...], w[...]) * dis[...]
    out0[...] = h[:, :HALF]
    out1[...] = h[:, HALF:]


def _tc_pre(x, w, dis):
    return pl.pallas_call(
        _pre_body,
        out_shape=(jax.ShapeDtypeStruct((N, HALF), jnp.float32),
                   jax.ShapeDtypeStruct((N, HALF), jnp.float32)),
    )(x, w, dis)


def _mid_body(agg, hp0, hp1, dis, b, w, out0, out1):
    hp = jnp.concatenate([hp0[...], hp1[...]], axis=1)
    o = (agg[...] + hp) * dis[...] + b[...]
    m = jnp.mean(o, axis=0, keepdims=True)
    cdev = o - m
    v = jnp.mean(cdev * cdev, axis=0, keepdims=True)
    y = _elu(cdev / jnp.sqrt(v + 1e-5))
    hn = jnp.dot(y, w[...]) * dis[...]
    out0[...] = hn[:, :HALF]
    out1[...] = hn[:, HALF:]


def _tc_mid(agg, hp0, hp1, dis, b, w):
    return pl.pallas_call(
        _mid_body,
        out_shape=(jax.ShapeDtypeStruct((N, HALF), jnp.float32),
                   jax.ShapeDtypeStruct((N, HALF), jnp.float32)),
    )(agg, hp0, hp1, dis, b, w)


def _post_body(agg, hp0, hp1, dis, b, out):
    hp = jnp.concatenate([hp0[...], hp1[...]], axis=1)
    o = (agg[...] + hp) * dis[...] + b[...]
    pooled = jnp.mean(o.reshape(50, N // 50, D), axis=1)
    out[...] = jnp.tanh(pooled)


def _tc_post(agg, hp0, hp1, dis, b, B):
    return pl.pallas_call(
        _post_body,
        out_shape=jax.ShapeDtypeStruct((B, D), jnp.float32),
    )(agg, hp0, hp1, dis, b)


def _head_body(c1, c2, e1, e2, wl0, bl0, wl1, bl1, ws0, bs0, ws1, bs1, ws2, bs2, out):
    d1 = jnp.concatenate([c1[...], e1[...]], axis=1)
    d2 = jnp.concatenate([c2[...], e2[...]], axis=1)

    def mlp1(x):
        x = _elu(jnp.dot(x, wl0[...]) + bl0[...])
        return jnp.dot(x, wl1[...]) + bl1[...]

    X = jnp.concatenate([mlp1(d1), mlp1(d2)], axis=1)
    X = _elu(jnp.dot(X, ws0[...]) + bs0[...])
    X = _elu(jnp.dot(X, ws1[...]) + bs1[...])
    out[...] = jnp.dot(X, ws2[...]) + bs2[...]


def _head(c1, c2, e1, e2, params):
    B = c1.shape[0]
    args = [c1, c2, e1, e2,
            params['Wl'][0], params['bl'][0], params['Wl'][1], params['bl'][1],
            params['Ws'][0], params['bs'][0], params['Ws'][1], params['bs'][1],
            params['Ws'][2], params['bs'][2]]
    return pl.pallas_call(
        _head_body,
        out_shape=jax.ShapeDtypeStruct((B, 1), jnp.float32),
    )(*args)


def _branch(x, edata, zeros, nrow, Ws, bs, dis, B):
    hp0, hp1 = _tc_pre(x, Ws[0], dis)
    for i in range(2):
        agg = _sc_agg(hp0, hp1, edata, zeros, nrow)
        hp0, hp1 = _tc_mid(agg, hp0, hp1, dis, bs[i].reshape(1, D), Ws[i + 1])
    agg = _sc_agg(hp0, hp1, edata, zeros, nrow)
    return _tc_post(agg, hp0, hp1, dis, bs[2].reshape(1, D), B)


def _pack_edges(src, dst, w=None):
    pads = EPAD - E
    src = jnp.concatenate([src, jnp.zeros((pads,), src.dtype)])
    dst = jnp.concatenate([dst, jnp.full((pads,), N, dst.dtype)])
    cols = [src.reshape(EPAD // CH, CH), dst.reshape(EPAD // CH, CH)]
    if w is not None:
        w = jnp.concatenate([w, jnp.zeros((pads,), w.dtype)])
        cols.append(lax.bitcast_convert_type(w, jnp.int32).reshape(EPAD // CH, CH))
    return jnp.stack(cols, axis=1)


def kernel(Drug1_F, Drug2_F, Drug1_ADJ, Drug2_ADJ, EXP1, EXP2, EXP_ADJ, EXP_ADJ_WGT, syn, params):
    B = syn.shape[0]
    s1, d1 = Drug1_ADJ[0], Drug1_ADJ[1]
    s2, d2 = Drug2_ADJ[0], Drug2_ADJ[1]
    se, de = EXP_ADJ[0], EXP_ADJ[1]
    parts = _sc_deg(d1, d2, de, EXP_ADJ_WGT)
    dis3 = _tc_dis(parts)
    dis_1 = dis3[0].reshape(N, 1)
    dis_2 = dis3[1].reshape(N, 1)
    dis_e = dis3[2].reshape(N, 1)
    ed1 = _pack_edges(s1, d1)
    ed2 = _pack_edges(s2, d2)
    ede = _pack_edges(se, de, EXP_ADJ_WGT)
    zeros = jnp.zeros((RPT, HALF), jnp.float32)
    c1 = _branch(Drug1_F, ed1, zeros, 2, params['Wc'], params['bc'], dis_1, B)
    c2 = _branch(Drug2_F, ed2, zeros, 2, params['Wc'], params['bc'], dis_2, B)
    e1 = _branch(EXP1, ede, zeros, 3, params['We'], params['be'], dis_e, B)
    e2 = _branch(EXP2, ede, zeros, 3, params['We'], params['be'], dis_e, B)
    return _head(c1, c2, e1, e2, params)


# speedup vs baseline: 1.9693x; 1.9693x over previous
import functools

import jax
import jax.numpy as jnp
from jax import lax
from jax.experimental import pallas as pl
from jax.experimental.pallas import tpu as pltpu
from jax.experimental.pallas import tpu_sc as plsc

N = 10000
E = 320000
D = 128
NSUB = 16
NCORE = 2
HALF = D // 2
RPT = N // NSUB
RCH = 125
CH = 80
EPAD = E
EPT = EPAD // NSUB
EPW = E // (NSUB * NCORE)
DCH = 2000


def _mesh():
    return plsc.VectorSubcoreMesh(core_axis_name="c", subcore_axis_name="s")


_SC_PARAMS = pltpu.CompilerParams(
    use_tc_tiling_on_sc=False, needs_layout_passes=False)


def _deg_body(dsta, dstb, dstc, wexp, parts, acc, dvm, wvm):
    c = lax.axis_index("c")
    s = lax.axis_index("s")
    wid = s * NCORE + c
    ebase = wid * EPW
    zv = jnp.zeros((16,), jnp.float32)
    ones = jnp.ones((16,), jnp.float32)
    for a, dste in enumerate((dsta, dstb, dstc)):
        def zb(i, _):
            acc[pl.ds(i * 16, 16)] = zv
            return 0
        lax.fori_loop(0, N // 16, zb, 0)
        for chn in range(EPW // DCH):
            off = ebase + chn * DCH
            pltpu.sync_copy(dste.at[pl.ds(off, DCH)], dvm)
            if a == 2:
                pltpu.sync_copy(wexp.at[pl.ds(off, DCH)], wvm)

            def sca(k, _):
                idx = dvm[pl.ds(k * 16, 16)]
                vals = wvm[pl.ds(k * 16, 16)] if a == 2 else ones
                plsc.addupdate_scatter(acc, [idx], vals)
                return 0
            lax.fori_loop(0, DCH // 16, sca, 0)
        pltpu.sync_copy(acc, parts.at[a, wid])


def _sc_deg(dsta, dstb, dstc, wexp):
    k = pl.kernel(
        _deg_body,
        mesh=_mesh(),
        out_type=jax.ShapeDtypeStruct((3, NSUB * NCORE, N), jnp.float32),
        scratch_types=[
            pltpu.VMEM((N,), jnp.float32),
            pltpu.VMEM((DCH,), jnp.int32),
            pltpu.VMEM((DCH,), jnp.float32),
        ],
        compiler_params=_SC_PARAMS,
    )
    return k(dsta, dstb, dstc, wexp)


KB = 5
SETS = 2
NQ = KB * SETS
CPT = EPT // CH
NIT = CPT // NQ


def _agg_body(nrow, hp0, hp1, edata, zeros, out, spm_acc, ebuf, rows,
              esem, gsem0, gsem1, ssem0, ssem1):
    scale = nrow == 3
    c = lax.axis_index("c")
    s = lax.axis_index("s")
    row0 = s * RPT
    col0 = c * HALF
    gsem = (gsem0, gsem1)
    ssem = (ssem0, ssem1)
    pltpu.sync_copy(zeros, spm_acc.at[pl.ds(row0, RPT)])
    plsc.subcore_barrier()

    cbase = s * CPT

    def gathers(p):
        def fire(hp):
            for k in range(KB):
                q = p * KB + k
                pltpu.async_copy(hp.at[ebuf.at[q, 0]], rows.at[q], gsem[p])

        @pl.when(c == 0)
        def _():
            fire(hp0)

        @pl.when(c == 1)
        def _():
            fire(hp1)

    def drain_gathers(p):
        def drain(hp):
            for k in range(KB):
                q = p * KB + k
                pltpu.make_async_copy(hp.at[ebuf.at[q, 0]], rows.at[q],
                                      gsem[p]).wait()

        @pl.when(c == 0)
        def _():
            drain(hp0)

        @pl.when(c == 1)
        def _():
            drain(hp1)

    def scale_rows(p):
        for k in range(KB):
            q = p * KB + k

            def sg(g, _):
                wvec = plsc.bitcast(ebuf[q, 2, pl.ds(g * 16, 16)],
                                    jnp.float32)
                for lane in range(16):
                    kk = g * 16 + lane
                    wk = wvec[lane]
                    for f in range(HALF // 16):
                        rows[q, kk, pl.ds(f * 16, 16)] = (
                            rows[q, kk, pl.ds(f * 16, 16)] * wk)
                return 0
            lax.fori_loop(0, CH // 16, sg, 0)

    def it(j, _):
        cid0 = cbase + j * NQ
        pltpu.sync_copy(edata.at[pl.ds(cid0, NQ)], ebuf)
        for p in range(SETS):
            gathers(p)
        for p in range(SETS):
            drain_gathers(p)
            if scale:
                scale_rows(p)
            for k in range(KB):
                q = p * KB + k
                pltpu.async_copy(rows.at[q], spm_acc.at[ebuf.at[q, 1]],
                                 ssem[p], add=True)
        for p in range(SETS):
            for k in range(KB):
                q = p * KB + k
                pltpu.make_async_copy(rows.at[q], spm_acc.at[ebuf.at[q, 1]],
                                      ssem[p]).wait()
        return 0
    lax.fori_loop(0, NIT, it, 0)
    plsc.subcore_barrier()
    pltpu.sync_copy(spm_acc.at[pl.ds(row0, RPT)],
                    out.at[pl.ds(row0, RPT), pl.ds(col0, HALF)])


def _sc_agg(hp0, hp1, edata, zeros, nrow):
    body = functools.partial(_agg_body, nrow)
    k = pl.kernel(
        body,
        mesh=_mesh(),
        out_type=jax.ShapeDtypeStruct((N, D), jnp.float32),
        scratch_types=[
            pltpu.VMEM_SHARED((N + 16, HALF), jnp.float32),
            pltpu.VMEM((NQ, nrow, CH), jnp.int32),
            pltpu.VMEM((NQ, CH, HALF), jnp.float32),
            pltpu.SemaphoreType.DMA,
            pltpu.SemaphoreType.DMA,
            pltpu.SemaphoreType.DMA,
            pltpu.SemaphoreType.DMA,
            pltpu.SemaphoreType.DMA,
        ],
        compiler_params=_SC_PARAMS,
    )
    return k(hp0, hp1, edata, zeros)


def _elu(x):
    return jnp.where(x > 0, x, jnp.exp(x) - 1.0)


def _dis_body(parts, out):
    deg = 1.0 + jnp.sum(parts[...], axis=1)
    out[...] = lax.rsqrt(deg)


def _tc_dis(parts):
    return pl.pallas_call(
        _dis_body,
        out_shape=jax.ShapeDtypeStruct((3, N), jnp.float32),
    )(parts)


def _pre_body(x, w, dis, out0, out1):
    h = jnp.dot(x[...], w[...]) * dis[...]
    out0[...] = h[:, :HALF]
    out1[...] = h[:, HALF:]


def _tc_pre(x, w, dis):
    return pl.pallas_call(
        _pre_body,
        out_shape=(jax.ShapeDtypeStruct((N, HALF), jnp.float32),
                   jax.ShapeDtypeStruct((N, HALF), jnp.float32)),
    )(x, w, dis)


def _mid_body(agg, hp0, hp1, dis, b, w, out0, out1):
    hp = jnp.concatenate([hp0[...], hp1[...]], axis=1)
    o = (agg[...] + hp) * dis[...] + b[...]
    m = jnp.mean(o, axis=0, keepdims=True)
    cdev = o - m
    v = jnp.mean(cdev * cdev, axis=0, keepdims=True)
    y = _elu(cdev / jnp.sqrt(v + 1e-5))
    hn = jnp.dot(y, w[...]) * dis[...]
    out0[...] = hn[:, :HALF]
    out1[...] = hn[:, HALF:]


def _tc_mid(agg, hp0, hp1, dis, b, w):
    return pl.pallas_call(
        _mid_body,
        out_shape=(jax.ShapeDtypeStruct((N, HALF), jnp.float32),
                   jax.ShapeDtypeStruct((N, HALF), jnp.float32)),
    )(agg, hp0, hp1, dis, b, w)


def _post_body(agg, hp0, hp1, dis, b, out):
    hp = jnp.concatenate([hp0[...], hp1[...]], axis=1)
    o = (agg[...] + hp) * dis[...] + b[...]
    pooled = jnp.mean(o.reshape(50, N // 50, D), axis=1)
    out[...] = jnp.tanh(pooled)


def _tc_post(agg, hp0, hp1, dis, b, B):
    return pl.pallas_call(
        _post_body,
        out_shape=jax.ShapeDtypeStruct((B, D), jnp.float32),
    )(agg, hp0, hp1, dis, b)


def _head_body(c1, c2, e1, e2, wl0, bl0, wl1, bl1, ws0, bs0, ws1, bs1, ws2, bs2, out):
    d1 = jnp.concatenate([c1[...], e1[...]], axis=1)
    d2 = jnp.concatenate([c2[...], e2[...]], axis=1)

    def mlp1(x):
        x = _elu(jnp.dot(x, wl0[...]) + bl0[...])
        return jnp.dot(x, wl1[...]) + bl1[...]

    X = jnp.concatenate([mlp1(d1), mlp1(d2)], axis=1)
    X = _elu(jnp.dot(X, ws0[...]) + bs0[...])
    X = _elu(jnp.dot(X, ws1[...]) + bs1[...])
    out[...] = jnp.dot(X, ws2[...]) + bs2[...]


def _head(c1, c2, e1, e2, params):
    B = c1.shape[0]
    args = [c1, c2, e1, e2,
            params['Wl'][0], params['bl'][0], params['Wl'][1], params['bl'][1],
            params['Ws'][0], params['bs'][0], params['Ws'][1], params['bs'][1],
            params['Ws'][2], params['bs'][2]]
    return pl.pallas_call(
        _head_body,
        out_shape=jax.ShapeDtypeStruct((B, 1), jnp.float32),
    )(*args)


def _branch(x, edata, zeros, nrow, Ws, bs, dis, B):
    hp0, hp1 = _tc_pre(x, Ws[0], dis)
    for i in range(2):
        agg = _sc_agg(hp0, hp1, edata, zeros, nrow)
        hp0, hp1 = _tc_mid(agg, hp0, hp1, dis, bs[i].reshape(1, D), Ws[i + 1])
    agg = _sc_agg(hp0, hp1, edata, zeros, nrow)
    return _tc_post(agg, hp0, hp1, dis, bs[2].reshape(1, D), B)


def _pack_edges(src, dst, w=None):
    cols = [src.reshape(EPAD // CH, CH), dst.reshape(EPAD // CH, CH)]
    if w is not None:
        cols.append(lax.bitcast_convert_type(w, jnp.int32).reshape(EPAD // CH, CH))
    return jnp.stack(cols, axis=1)


def kernel(Drug1_F, Drug2_F, Drug1_ADJ, Drug2_ADJ, EXP1, EXP2, EXP_ADJ, EXP_ADJ_WGT, syn, params):
    B = syn.shape[0]
    s1, d1 = Drug1_ADJ[0], Drug1_ADJ[1]
    s2, d2 = Drug2_ADJ[0], Drug2_ADJ[1]
    se, de = EXP_ADJ[0], EXP_ADJ[1]
    parts = _sc_deg(d1, d2, de, EXP_ADJ_WGT)
    dis3 = _tc_dis(parts)
    dis_1 = dis3[0].reshape(N, 1)
    dis_2 = dis3[1].reshape(N, 1)
    dis_e = dis3[2].reshape(N, 1)
    ed1 = _pack_edges(s1, d1)
    ed2 = _pack_edges(s2, d2)
    ede = _pack_edges(se, de, EXP_ADJ_WGT)
    zeros = jnp.zeros((RPT, HALF), jnp.float32)
    c1 = _branch(Drug1_F, ed1, zeros, 2, params['Wc'], params['bc'], dis_1, B)
    c2 = _branch(Drug2_F, ed2, zeros, 2, params['Wc'], params['bc'], dis_2, B)
    e1 = _branch(EXP1, ede, zeros, 3, params['We'], params['be'], dis_e, B)
    e2 = _branch(EXP2, ede, zeros, 3, params['We'], params['be'], dis_e, B)
    return _head(c1, c2, e1, e2, params)
